# baseline (speedup 1.0000x reference)
import jax
import jax.numpy as jnp
from jax import lax
from jax.experimental import pallas as pl
from jax.experimental.pallas import tpu as pltpu
from jax.experimental.pallas import tpu_sc as plsc

HIDDEN = 64
WIDE = 128
LANES = 16
NC, NS = 2, 16
NW = NC * NS
CHUNK = 128
NB = 2
UNROLL = 4
SCALE = 8.0


def _build(B, S):
    T = B * S
    assert T % (NW * CHUNK) == 0
    tpw = T // NW
    steps = tpw // CHUNK
    assert steps > NB
    mesh = plsc.VectorSubcoreMesh(
        core_axis_name="c", subcore_axis_name="s",
        num_cores=NC, num_subcores=NS)

    def body(x_hbm, table_hbm, out_hbm, idx_v, gbuf, sbuf, gsem, ssem):
        wid = lax.axis_index("s") * NC + lax.axis_index("c")
        tbase = wid * tpw

        def fire_gather(c):
            pltpu.async_copy(
                table_hbm.at[idx_v.at[pl.ds(c * CHUNK, CHUNK)]],
                gbuf.at[c % NB], gsem.at[c % NB])

        def wait_gather(c):
            pltpu.make_async_copy(
                table_hbm.at[idx_v.at[pl.ds(c * CHUNK, CHUNK)]],
                gbuf.at[c % NB], gsem.at[c % NB]).wait()

        def fire_store(c):
            pltpu.async_copy(
                sbuf.at[c % NB],
                out_hbm.at[pl.ds(tbase + c * CHUNK, CHUNK)],
                ssem.at[c % NB])

        def wait_store(c):
            pltpu.make_async_copy(
                sbuf.at[c % NB],
                out_hbm.at[pl.ds(tbase + c * CHUNK, CHUNK)],
                ssem.at[c % NB]).wait()

        pltpu.sync_copy(x_hbm.at[pl.ds(tbase, tpw)], idx_v)

        for c in range(NB):
            fire_gather(c)

        @pl.loop(0, steps)
        def _step(c):
            b = c % NB
            wait_gather(c)

            @pl.when(c >= NB)
            def _():
                wait_store(c - NB)

            @pl.loop(0, CHUNK // UNROLL)
            def _tok(u):
                for k in range(UNROLL):
                    t = u * UNROLL + k
                    for j in range(HIDDEN // LANES):
                        sl = pl.ds(j * LANES, LANES)
                        sbuf[b, t, sl] = gbuf[b, t, sl]

            fire_store(c)

            @pl.when(c + NB < steps)
            def _():
                fire_gather(c + NB)

        for c in range(steps - NB, steps):
            wait_store(c)

    return pl.kernel(
        body,
        out_type=jax.ShapeDtypeStruct((T, HIDDEN), jnp.float32),
        mesh=mesh,
        scratch_types=[
            pltpu.VMEM((tpw,), jnp.int32),
            pltpu.VMEM((NB, CHUNK, WIDE), jnp.float32),
            pltpu.VMEM((NB, CHUNK, HIDDEN), jnp.float32),
            pltpu.SemaphoreType.DMA((NB,)),
            pltpu.SemaphoreType.DMA((NB,)),
        ],
        compiler_params=pltpu.CompilerParams(use_tc_tiling_on_sc=True),
    )


def kernel(x, table):
    b, s = x.shape
    v, h = table.shape
    table_p = jnp.pad(table * SCALE, ((0, 0), (0, WIDE - h)))
    x_flat = x.astype(jnp.int32).reshape(b * s)
    out = _build(b, s)(x_flat, table_p)
    return out.reshape(b, s, h)


# speedup vs baseline: 1.0948x; 1.0948x over previous
import jax
import jax.numpy as jnp
from jax import lax
from jax.experimental import pallas as pl
from jax.experimental.pallas import tpu as pltpu
from jax.experimental.pallas import tpu_sc as plsc

HIDDEN = 64
WIDE = 128
LANES = 16
NC, NS = 2, 16
NW = NC * NS
CHUNK = 128
NB = 2
SCALE = 8.0


def _build(B, S):
    T = B * S
    assert T % (NW * CHUNK) == 0
    tpw = T // NW
    steps = tpw // CHUNK
    assert steps % NB == 0
    mesh = plsc.VectorSubcoreMesh(
        core_axis_name="c", subcore_axis_name="s",
        num_cores=NC, num_subcores=NS)

    def body(x_hbm, table_hbm, out_hbm, idx_v, gbuf, sbuf, gsem, ssem):
        wid = lax.axis_index("s") * NC + lax.axis_index("c")
        tbase = wid * tpw

        def fire_gather(b, c):
            pltpu.async_copy(
                table_hbm.at[idx_v.at[pl.ds(c * CHUNK, CHUNK)]],
                gbuf.at[b], gsem.at[b])

        def wait_gather(b, c):
            pltpu.make_async_copy(
                table_hbm.at[idx_v.at[pl.ds(c * CHUNK, CHUNK)]],
                gbuf.at[b], gsem.at[b]).wait()

        def fire_store(b, c):
            pltpu.async_copy(
                sbuf.at[b],
                out_hbm.at[pl.ds(tbase + c * CHUNK, CHUNK)],
                ssem.at[b])

        def wait_store(b):
            pltpu.make_async_copy(
                sbuf.at[b],
                out_hbm.at[pl.ds(tbase, CHUNK)],
                ssem.at[b]).wait()

        pltpu.sync_copy(x_hbm.at[pl.ds(tbase, tpw)], idx_v)

        for b in range(NB):
            fire_gather(b, b)

        @pl.loop(0, steps // NB)
        def _step(step):
            for b in range(NB):
                c = step * NB + b
                wait_gather(b, c)

                @pl.when(step > 0)
                def _():
                    wait_store(b)

                @pl.loop(0, CHUNK)
                def _tok(t):
                    for j in range(HIDDEN // LANES):
                        sl = pl.ds(j * LANES, LANES)
                        sbuf[b, t, sl] = gbuf[b, t, sl]

                fire_store(b, c)

                @pl.when(step < steps // NB - 1)
                def _():
                    fire_gather(b, c + NB)

        for b in range(NB):
            wait_store(b)

    return pl.kernel(
        body,
        out_type=jax.ShapeDtypeStruct((T, HIDDEN), jnp.float32),
        mesh=mesh,
        scratch_types=[
            pltpu.VMEM((tpw,), jnp.int32),
            pltpu.VMEM((NB, CHUNK, WIDE), jnp.float32),
            pltpu.VMEM((NB, CHUNK, HIDDEN), jnp.float32),
            pltpu.SemaphoreType.DMA((NB,)),
            pltpu.SemaphoreType.DMA((NB,)),
        ],
        compiler_params=pltpu.CompilerParams(use_tc_tiling_on_sc=True),
    )


def kernel(x, table):
    b, s = x.shape
    v, h = table.shape
    table_p = jnp.pad(table * SCALE, ((0, 0), (0, WIDE - h)))
    x_flat = x.astype(jnp.int32).reshape(b * s)
    out = _build(b, s)(x_flat, table_p)
    return out.reshape(b, s, h)
